# baseline (speedup 1.0000x reference)
import jax
import jax.numpy as jnp
from jax.experimental import pallas as pl
from jax.experimental.pallas import tpu as pltpu

OBS_DIM = 55
A1_DIM = 10
A2_DIM = 10
IN_DIM = OBS_DIM + A1_DIM + A2_DIM
HIDDEN = 128
OUT_DIM = 35
NEG_SLOPE = 0.01

R = 8
_TILE_M = 2048


def _mlp_kernel(x_ref,
                w1_ref, b1_ref,
                w2_ref, b2_ref,
                w3_ref, b3_ref,
                o_ref):
    f32 = jnp.float32
    bf16 = jnp.bfloat16
    h = (jnp.dot(x_ref[...], w1_ref[...], preferred_element_type=f32)
         + b1_ref[...])

    h = jnp.dot(h.astype(bf16), w2_ref[...],
                preferred_element_type=f32) + b2_ref[...]
    h = jnp.where(h >= 0, h, NEG_SLOPE * h)

    o_ref[...] = (jnp.dot(h.astype(bf16), w3_ref[...],
                          preferred_element_type=f32)
                  + b3_ref[...]).astype(o_ref.dtype)


def _block_diag(w, r):
    return jnp.kron(jnp.eye(r, dtype=w.dtype), w)


def kernel(observation, action_j1, action_j2, w1o, w1a, b1, w2, b2, w3, b3):
    B = observation.shape[0]
    bf16 = jnp.bfloat16
    f32 = jnp.float32

    w1 = jnp.concatenate([w1o, w1a], axis=0)
    w1_c = _block_diag(w1.astype(bf16), R)
    w2_c = _block_diag(w2.astype(bf16), R)
    w3_c = _block_diag(w3.astype(bf16), R)
    b1_c = jnp.tile(b1.astype(f32), (1, R))
    b2_c = jnp.tile(b2.astype(f32), (1, R))
    b3_c = jnp.tile(b3.astype(f32), (1, R))

    x = jnp.concatenate([observation, action_j1, action_j2],
                        axis=1).astype(bf16)

    rows = pl.cdiv(B, R)
    n_steps = max(2, pl.cdiv(rows, _TILE_M))
    tile_m = ((pl.cdiv(rows, n_steps) + 7) // 8) * 8
    rows_p = n_steps * tile_m
    Bp = rows_p * R
    if Bp != B:
        x = jnp.pad(x, ((0, Bp - B), (0, 0)))
    x_m = x.reshape(rows_p, R * IN_DIM)

    def batch_spec(feat):
        return pl.BlockSpec((tile_m, feat), lambda i: (i, 0))

    def resident_spec(arr):
        return pl.BlockSpec(arr.shape, lambda i: (0, 0))

    weight_bytes = (2 * (w1_c.size + w2_c.size + w3_c.size)
                    + 4 * (b1_c.size + b2_c.size + b3_c.size))
    cost = pl.CostEstimate(
        flops=2 * rows_p * R * R * (IN_DIM * HIDDEN + HIDDEN * HIDDEN
                                    + HIDDEN * OUT_DIM),
        transcendentals=0,
        bytes_accessed=Bp * (2 * IN_DIM + 4 * OUT_DIM) + weight_bytes)

    out_m = pl.pallas_call(
        _mlp_kernel,
        out_shape=jax.ShapeDtypeStruct((rows_p, R * OUT_DIM), f32),
        grid=(n_steps,),
        in_specs=[
            batch_spec(R * IN_DIM),
            resident_spec(w1_c), resident_spec(b1_c),
            resident_spec(w2_c), resident_spec(b2_c),
            resident_spec(w3_c), resident_spec(b3_c),
        ],
        out_specs=batch_spec(R * OUT_DIM),
        compiler_params=pltpu.CompilerParams(
            dimension_semantics=("parallel",)),
        cost_estimate=cost,
    )(x_m, w1_c, b1_c, w2_c, b2_c, w3_c, b3_c)

    out = out_m.reshape(Bp, OUT_DIM)
    return out[:B] if Bp != B else out


# speedup vs baseline: 1.3994x; 1.3994x over previous
import jax
import jax.numpy as jnp
from jax.experimental import pallas as pl
from jax.experimental.pallas import tpu as pltpu

OBS_DIM = 55
A1_DIM = 10
A2_DIM = 10
IN_DIM = OBS_DIM + A1_DIM + A2_DIM
HIDDEN = 128
OUT_DIM = 35
NEG_SLOPE = 0.01

_TILE_B = 16384
_SINGLE_STEP_MAX_B = 511


def _mlp_kernel(x_ref,
                w1_ref, b1_ref,
                w2_ref, b2_ref,
                w3_ref, b3_ref,
                o_ref):
    f32 = jnp.float32
    bf16 = jnp.bfloat16
    h = (jnp.dot(x_ref[...], w1_ref[...], preferred_element_type=f32)
         + b1_ref[...])

    h = jnp.dot(h.astype(bf16), w2_ref[...],
                preferred_element_type=f32) + b2_ref[...]
    h = jnp.where(h >= 0, h, NEG_SLOPE * h)

    o_ref[...] = (jnp.dot(h.astype(bf16), w3_ref[...],
                          preferred_element_type=f32)
                  + b3_ref[...]).astype(o_ref.dtype)


def _choose_tiling(B):
    if B <= _SINGLE_STEP_MAX_B:
        return 1, B
    n_steps = max(2, pl.cdiv(B, _TILE_B))
    tile_b = pl.cdiv(B, n_steps)
    tile_b = ((tile_b + 7) // 8) * 8
    return n_steps, tile_b


def kernel(observation, action_j1, action_j2, w1o, w1a, b1, w2, b2, w3, b3):
    B = observation.shape[0]
    bf16 = jnp.bfloat16
    f32 = jnp.float32

    w1_c = jnp.concatenate(
        [w1o, w1a, jnp.zeros((128 - IN_DIM, HIDDEN), w1o.dtype)],
        axis=0).astype(bf16)
    w2_c = w2.astype(bf16)
    w3_c = w3.astype(bf16)
    b1_c = b1.astype(f32)
    b2_c = b2.astype(f32)
    b3_c = b3.astype(f32)

    x = jnp.concatenate(
        [observation, action_j1, action_j2,
         jnp.zeros((B, 128 - IN_DIM), observation.dtype)],
        axis=1).astype(bf16)

    n_steps, tile_b = _choose_tiling(B)
    Bp = n_steps * tile_b
    pad = Bp - B
    if pad:
        x = jnp.pad(x, ((0, pad), (0, 0)))

    def batch_spec(feat):
        return pl.BlockSpec((tile_b, feat), lambda i: (i, 0))

    def resident_spec(arr):
        return pl.BlockSpec(arr.shape, lambda i: (0, 0))

    weight_bytes = (2 * (w1_c.size + w2_c.size + w3_c.size)
                    + 4 * (b1_c.size + b2_c.size + b3_c.size))
    cost = pl.CostEstimate(
        flops=2 * Bp * (IN_DIM * HIDDEN + HIDDEN * HIDDEN + HIDDEN * OUT_DIM),
        transcendentals=0,
        bytes_accessed=Bp * (2 * IN_DIM + 4 * OUT_DIM) + weight_bytes)

    out = pl.pallas_call(
        _mlp_kernel,
        out_shape=jax.ShapeDtypeStruct((Bp, OUT_DIM), f32),
        grid=(n_steps,),
        in_specs=[
            batch_spec(128),
            resident_spec(w1_c), resident_spec(b1_c),
            resident_spec(w2_c), resident_spec(b2_c),
            resident_spec(w3_c), resident_spec(b3_c),
        ],
        out_specs=batch_spec(OUT_DIM),
        compiler_params=pltpu.CompilerParams(
            dimension_semantics=("parallel",)),
        cost_estimate=cost,
    )(x, w1_c, b1_c, w2_c, b2_c, w3_c, b3_c)

    return out[:B] if pad else out
